# Initial kernel scaffold; baseline (speedup 1.0000x reference)
import jax
import jax.numpy as jnp
from jax.experimental import pallas as pl


def kernel(x, edge_index, W1, att_src1, att_dst1, b1, W2, att_src2, att_dst2, b2):
    raise NotImplementedError("write your pallas kernel here")


# speedup vs baseline: 22.5747x; 22.5747x over previous
import functools

import jax
import jax.numpy as jnp
from jax import lax
from jax.experimental import pallas as pl
from jax.experimental.pallas import tpu as pltpu
from jax.experimental.pallas import tpu_sc as plsc

N = 10000
D = 128
DA = 144
E_RAW = 320000
E = E_RAW + N
NC, NS, LANES = 2, 16, 16
NW = NC * NS
BLK = 128
EPT = -(-E // (NW * BLK)) * BLK
E_PAD = EPT * NW
NBLK = EPT // BLK
ROWS_PER_TILE = N // NS

_R = 1000


def _dense_body(x_ref, w_ref, as_ref, ad_ref, haug_ref, asrc_ref, adst_ref):
    h = jnp.dot(x_ref[...], w_ref[...], preferred_element_type=jnp.float32)
    haug_ref[:, :D] = h
    haug_ref[:, D:] = jnp.zeros((_R, DA - D), jnp.float32)
    asrc_ref[...] = jnp.sum(h * as_ref[...][None, :], axis=1, keepdims=True)
    adst_ref[...] = jnp.sum(h * ad_ref[...][None, :], axis=1, keepdims=True)


def _dense(x, w, att_src, att_dst):
    grid = (N // _R,)
    return pl.pallas_call(
        _dense_body,
        grid=grid,
        in_specs=[
            pl.BlockSpec((_R, D), lambda i: (i, 0)),
            pl.BlockSpec((D, D), lambda i: (0, 0)),
            pl.BlockSpec((D,), lambda i: (0,)),
            pl.BlockSpec((D,), lambda i: (0,)),
        ],
        out_specs=[
            pl.BlockSpec((_R, DA), lambda i: (i, 0)),
            pl.BlockSpec((_R, 1), lambda i: (i, 0)),
            pl.BlockSpec((_R, 1), lambda i: (i, 0)),
        ],
        out_shape=[
            jax.ShapeDtypeStruct((N, DA), jnp.float32),
            jax.ShapeDtypeStruct((N, 1), jnp.float32),
            jax.ShapeDtypeStruct((N, 1), jnp.float32),
        ],
    )(x, w, att_src, att_dst)


def _combine_body(do_relu, acca_ref, accb_ref, b_ref, out_ref):
    s = acca_ref[...] + accb_ref[...]
    out = s[:, :D] / (s[:, D][:, None] + 1e-16) + b_ref[...][None, :]
    if do_relu:
        out = jnp.maximum(out, 0.0)
    out_ref[...] = out


def _combine(acca, accb, b, do_relu):
    return pl.pallas_call(
        functools.partial(_combine_body, do_relu),
        grid=(N // _R,),
        in_specs=[
            pl.BlockSpec((_R, DA), lambda i: (i, 0)),
            pl.BlockSpec((_R, DA), lambda i: (i, 0)),
            pl.BlockSpec((D,), lambda i: (0,)),
        ],
        out_specs=pl.BlockSpec((_R, D), lambda i: (i, 0)),
        out_shape=jax.ShapeDtypeStruct((N, D), jnp.float32),
    )(acca, accb, b)


def _sc_edge_body(haug, asrc, adst, src, dst, zeros, acca, accb,
                  asrc_v, adst_v, src_v, dst_v, ex_v, rows_v, acc_sh, sem):
    cid = lax.axis_index("c")
    sid = lax.axis_index("s")
    base = (cid * NS + sid) * EPT

    pltpu.sync_copy(asrc, asrc_v)
    pltpu.sync_copy(adst, adst_v)
    pltpu.sync_copy(zeros, acc_sh.at[pl.ds(sid * ROWS_PER_TILE, ROWS_PER_TILE)])
    plsc.subcore_barrier()

    @pl.loop(0, NBLK)
    def _blk(blk):
        ebase = base + blk * BLK
        pltpu.sync_copy(src.at[pl.ds(ebase, BLK)], src_v)
        pltpu.sync_copy(dst.at[pl.ds(ebase, BLK)], dst_v)
        gather = pltpu.async_copy(haug.at[src_v], rows_v, sem)

        @pl.loop(0, BLK, step=LANES)
        def _ex(i):
            s_idx = src_v[pl.ds(i, LANES)]
            d_idx = dst_v[pl.ds(i, LANES)]
            e = plsc.load_gather(asrc_v, [s_idx]) + plsc.load_gather(adst_v, [d_idx])
            e = jnp.where(e >= 0.0, e, 0.2 * e)
            ex = jnp.exp(e)
            g = ebase + i + lax.iota(jnp.int32, LANES)
            ex_v[pl.ds(i, LANES)] = jnp.where(g < E, ex, 0.0)

        gather.wait()

        ones0 = jnp.where(lax.iota(jnp.int32, LANES) == 0, 1.0, 0.0)

        @pl.loop(0, BLK, step=LANES)
        def _scale(g):
            exv = ex_v[pl.ds(g, LANES)]
            for k in range(LANES):
                i = g + k
                s = exv[k]
                for j in range(D // LANES):
                    sl = pl.ds(j * LANES, LANES)
                    rows_v[i, sl] = rows_v[i, sl] * s
                rows_v[i, pl.ds(D, LANES)] = ones0 * s

        pltpu.sync_copy(rows_v, acc_sh.at[dst_v], add=True)

    plsc.subcore_barrier()
    rs = sid * ROWS_PER_TILE

    @pl.when(cid == 0)
    def _():
        pltpu.sync_copy(acc_sh.at[pl.ds(rs, ROWS_PER_TILE)],
                        acca.at[pl.ds(rs, ROWS_PER_TILE)])

    @pl.when(cid == 1)
    def _():
        pltpu.sync_copy(acc_sh.at[pl.ds(rs, ROWS_PER_TILE)],
                        accb.at[pl.ds(rs, ROWS_PER_TILE)])


def _sc_edge(haug, asrc, adst, src, dst, zeros):
    mesh = plsc.VectorSubcoreMesh(core_axis_name="c", subcore_axis_name="s")
    f = pl.kernel(
        _sc_edge_body,
        out_type=[
            jax.ShapeDtypeStruct((N, DA), jnp.float32),
            jax.ShapeDtypeStruct((N, DA), jnp.float32),
        ],
        mesh=mesh,
        scratch_types=[
            pltpu.VMEM((N,), jnp.float32),
            pltpu.VMEM((N,), jnp.float32),
            pltpu.VMEM((BLK,), jnp.int32),
            pltpu.VMEM((BLK,), jnp.int32),
            pltpu.VMEM((BLK,), jnp.float32),
            pltpu.VMEM((BLK, DA), jnp.float32),
            pltpu.VMEM_SHARED((N, DA), jnp.float32),
            pltpu.SemaphoreType.DMA,
        ],
        compiler_params=pltpu.CompilerParams(use_tc_tiling_on_sc=False,
                                             needs_layout_passes=False),
    )
    return f(haug, asrc, adst, src, dst, zeros)


def kernel(x, edge_index, W1, att_src1, att_dst1, b1, W2, att_src2, att_dst2, b2):
    loop = jnp.arange(N, dtype=jnp.int32)
    pad = jnp.zeros((E_PAD - E,), jnp.int32)
    src = jnp.concatenate([edge_index[0].astype(jnp.int32), loop, pad])
    dst = jnp.concatenate([edge_index[1].astype(jnp.int32), loop, pad])
    zeros = jnp.zeros((ROWS_PER_TILE, DA), jnp.float32)

    haug1, a1s, a1d = _dense(x, W1, att_src1, att_dst1)
    acca1, accb1 = _sc_edge(haug1, a1s.reshape(N), a1d.reshape(N), src, dst, zeros)
    emb = _combine(acca1, accb1, b1, do_relu=True)

    haug2, a2s, a2d = _dense(emb, W2, att_src2, att_dst2)
    acca2, accb2 = _sc_edge(haug2, a2s.reshape(N), a2d.reshape(N), src, dst, zeros)
    out = _combine(acca2, accb2, b2, do_relu=False)
    return (emb, out)
